# Initial kernel scaffold; baseline (speedup 1.0000x reference)
import jax
import jax.numpy as jnp
from jax.experimental import pallas as pl


def kernel(x):
    raise NotImplementedError("write your pallas kernel here")


# speedup vs baseline: 5.0464x; 5.0464x over previous
import functools

import jax
import jax.numpy as jnp
from jax import lax
from jax.experimental import pallas as pl
from jax.experimental.pallas import tpu as pltpu
from jax.experimental.pallas import tpu_sc as plsc

_GS = 128
_LANES = 16
_VPG = _GS // _LANES
_CH = 256
_U = 4


def _sort16(v):
    return lax.sort(v, dimension=0)


def _rev(v):
    return lax.rev(v, (0,))


def _merge(a, b):
    m = len(a)
    c = a + [_rev(b[m - 1 - i]) for i in range(m)]
    stride = m
    while stride >= 1:
        nxt = list(c)
        for base in range(0, 2 * m, 2 * stride):
            for i in range(stride):
                lo, hi = c[base + i], c[base + stride + i]
                nxt[base + i] = jnp.minimum(lo, hi)
                nxt[base + stride + i] = jnp.maximum(lo, hi)
        c = nxt
        stride //= 2
    return [_sort16(v) for v in c]


def _sort_group(vs):
    runs = [[_sort16(v)] for v in vs]
    while len(runs) > 1:
        runs = [_merge(runs[2 * i], runs[2 * i + 1])
                for i in range(len(runs) // 2)]
    return runs[0]


@functools.lru_cache(maxsize=None)
def _build(rows):
    info = plsc.get_sparse_core_info()
    nc, ns = info.num_cores, info.num_subcores
    nw = nc * ns
    rpw = rows // nw
    ch = min(_CH, rpw)
    mesh = plsc.VectorSubcoreMesh(core_axis_name="c", subcore_axis_name="s")

    @functools.partial(
        pl.kernel,
        mesh=mesh,
        out_type=jax.ShapeDtypeStruct((rows, _GS), jnp.float32),
        scratch_types=[pltpu.VMEM((ch, _GS), jnp.float32)],
        compiler_params=pltpu.CompilerParams(needs_layout_passes=False),
    )
    def sc_group_sort(x_hbm, out_hbm, buf):
        wid = lax.axis_index("s") * nc + lax.axis_index("c")
        row0 = wid * rpw
        for cblk in range(rpw // ch):
            base = row0 + cblk * ch

            pltpu.sync_copy(x_hbm.at[pl.ds(base, ch)], buf)

            def body(i, carry):
                for u in range(_U):
                    g = i * _U + u
                    vs = [buf[g, pl.ds(j * _LANES, _LANES)]
                          for j in range(_VPG)]
                    sv = _sort_group(vs)
                    for j in range(_VPG):
                        buf[g, pl.ds(j * _LANES, _LANES)] = sv[j]
                return carry

            lax.fori_loop(0, ch // _U, body, 0)

            pltpu.sync_copy(buf, out_hbm.at[pl.ds(base, ch)])

    return sc_group_sort


def kernel(x):
    b, f = x.shape
    rows = b * f // _GS
    xr = x.reshape(rows, _GS)
    out = _build(rows)(xr)
    return out.reshape(b, f)


# speedup vs baseline: 5.5771x; 1.1052x over previous
import functools

import jax
import jax.numpy as jnp
from jax import lax
from jax.experimental import pallas as pl
from jax.experimental.pallas import tpu as pltpu
from jax.experimental.pallas import tpu_sc as plsc

_GS = 128
_LANES = 16
_VPG = _GS // _LANES
_CH = 256
_U = 4


def _sort16(v):
    return lax.sort(v, dimension=0)


def _rev(v):
    return lax.rev(v, (0,))


def _merge(a, b):
    m = len(a)
    c = a + [_rev(b[m - 1 - i]) for i in range(m)]
    stride = m
    while stride >= 1:
        nxt = list(c)
        for base in range(0, 2 * m, 2 * stride):
            for i in range(stride):
                lo, hi = c[base + i], c[base + stride + i]
                nxt[base + i] = jnp.minimum(lo, hi)
                nxt[base + stride + i] = jnp.maximum(lo, hi)
        c = nxt
        stride //= 2
    return [_sort16(v) for v in c]


def _sort_group(vs):
    runs = [[_sort16(v)] for v in vs]
    while len(runs) > 1:
        runs = [_merge(runs[2 * i], runs[2 * i + 1])
                for i in range(len(runs) // 2)]
    return runs[0]


@functools.lru_cache(maxsize=None)
def _build(rows):
    info = plsc.get_sparse_core_info()
    nc, ns = info.num_cores, info.num_subcores
    nw = nc * ns
    rpw = rows // nw
    ch = min(_CH, rpw)
    mesh = plsc.VectorSubcoreMesh(core_axis_name="c", subcore_axis_name="s")

    nch = rpw // ch

    @functools.partial(
        pl.kernel,
        mesh=mesh,
        out_type=jax.ShapeDtypeStruct((rows, _GS), jnp.float32),
        scratch_types=[
            pltpu.VMEM((ch, _GS), jnp.float32),
            pltpu.VMEM((ch, _GS), jnp.float32),
            pltpu.SemaphoreType.DMA,
            pltpu.SemaphoreType.DMA,
            pltpu.SemaphoreType.DMA,
            pltpu.SemaphoreType.DMA,
        ],
        compiler_params=pltpu.CompilerParams(needs_layout_passes=False),
    )
    def sc_group_sort(x_hbm, out_hbm, b0, b1, si0, si1, so0, so1):
        wid = lax.axis_index("s") * nc + lax.axis_index("c")
        row0 = wid * rpw
        bufs, sin, sout = [b0, b1], [si0, si1], [so0, so1]

        def start_in(c):
            base = row0 + c * ch
            return pltpu.async_copy(
                x_hbm.at[pl.ds(base, ch)], bufs[c % 2], sin[c % 2])

        def start_out(c):
            base = row0 + c * ch
            return pltpu.async_copy(
                bufs[c % 2], out_hbm.at[pl.ds(base, ch)], sout[c % 2])

        in_h = {0: start_in(0)}
        out_h = {}
        for c in range(nch):
            if c + 1 < nch:
                if c - 1 >= 0:
                    out_h[c - 1].wait()
                in_h[c + 1] = start_in(c + 1)
            in_h[c].wait()
            buf = bufs[c % 2]

            @plsc.parallel_loop(0, ch, step=1, unroll=_U)
            def body(g):
                vs = [buf[g, pl.ds(j * _LANES, _LANES)]
                      for j in range(_VPG)]
                sv = _sort_group(vs)
                for j in range(_VPG):
                    buf[g, pl.ds(j * _LANES, _LANES)] = sv[j]

            out_h[c] = start_out(c)
        for c in range(max(0, nch - 2), nch):
            out_h[c].wait()

    return sc_group_sort


def kernel(x):
    b, f = x.shape
    rows = b * f // _GS
    xr = x.reshape(rows, _GS)
    out = _build(rows)(xr)
    return out.reshape(b, f)
